# baseline (speedup 1.0000x reference)
import functools

import jax
import jax.numpy as jnp
from jax.experimental import pallas as pl
from jax.experimental.pallas import tpu as pltpu

N = 10000
E = 320000
HEADS = 8
HID = 128


def _mlp_body(h_ref, w1_ref, b1_ref, w2_ref, b2_ref, o_ref):
    h = h_ref[...]
    z = jnp.maximum(jnp.dot(h, w1_ref[...], preferred_element_type=jnp.float32)
                    + b1_ref[...][None, :], 0.0)
    o_ref[...] = jnp.dot(z, w2_ref[...], preferred_element_type=jnp.float32) + b2_ref[...][None, :]


def _mlp(h, cls_W1, cls_b1, cls_W2, cls_b2):
    n = h.shape[0]
    blk = 2000
    w2p = jnp.zeros((64, 128), jnp.float32).at[:, :2].set(cls_W2)
    b2p = jnp.zeros((128,), jnp.float32).at[:2].set(cls_b2)
    out = pl.pallas_call(
        _mlp_body,
        grid=(n // blk,),
        in_specs=[
            pl.BlockSpec((blk, HID), lambda i: (i, 0)),
            pl.BlockSpec((HID, 64), lambda i: (0, 0)),
            pl.BlockSpec((64,), lambda i: (0,)),
            pl.BlockSpec((64, 128), lambda i: (0, 0)),
            pl.BlockSpec((128,), lambda i: (0,)),
        ],
        out_specs=pl.BlockSpec((blk, 128), lambda i: (i, 0)),
        out_shape=jax.ShapeDtypeStruct((n, 128), jnp.float32),
    )(h, cls_W1, cls_b1, w2p, b2p)
    return out[:, :2]


def _gat(x, edge_index, W, a_s, a_d, b, heads, out_ch, concat):
    n = x.shape[0]
    h = (x @ W).reshape(n, heads, out_ch)
    src = edge_index[0]
    dst = edge_index[1]
    al_s = jnp.sum(h * a_s[None, :, :], axis=-1)
    al_d = jnp.sum(h * a_d[None, :, :], axis=-1)
    e = al_s[src] + al_d[dst]
    e = jax.nn.leaky_relu(e, 0.2)
    m = jax.ops.segment_max(e, dst, num_segments=n)
    m = jnp.where(jnp.isfinite(m), m, 0.0)
    ex = jnp.exp(e - m[dst])
    denom = jax.ops.segment_sum(ex, dst, num_segments=n)
    alpha = ex / (denom[dst] + 1e-16)
    out = jax.ops.segment_sum(h[src] * alpha[:, :, None], dst, num_segments=n)
    if concat:
        out = out.reshape(n, heads * out_ch)
    else:
        out = out.mean(axis=1)
    return out + b


def _bn(x, g, b):
    mu = jnp.mean(x, axis=0)
    var = jnp.var(x, axis=0)
    return (x - mu) / jnp.sqrt(var + 1e-5) * g + b


def kernel(x, edge_index, W1, att_src1, att_dst1, bias1, gamma1, beta1, W2, att_src2, att_dst2, bias2, gamma2, beta2, cls_W1, cls_b1, cls_W2, cls_b2):
    h = _gat(x, edge_index, W1, att_src1, att_dst1, bias1, HEADS, HID, True)
    h = jax.nn.elu(_bn(h, gamma1, beta1))
    h = _gat(h, edge_index, W2, att_src2, att_dst2, bias2, 1, HID, False)
    h = jax.nn.elu(_bn(h, gamma2, beta2))
    return _mlp(h, cls_W1, cls_b1, cls_W2, cls_b2)


# speedup vs baseline: 12.9955x; 12.9955x over previous
import jax
import jax.numpy as jnp
from jax import lax
from jax.experimental import pallas as pl
from jax.experimental.pallas import tpu as pltpu
from jax.experimental.pallas import tpu_sc as plsc

N = 10000
E = 320000
HEADS = 8
HID = 128
FH = 64
NQ = 2
PWH = 80
BLK = 400
NB = N // BLK
KCH = 80
EPT1 = E // 16
NCH1 = EPT1 // KCH
EPT2 = E // 32
NCH2 = EPT2 // KCH
NACC = 10000
RTILES = 10
RZ = NACC // RTILES
NT = 10240
BLKP = 512
NBP = NT // BLKP

_SC_PARAMS = pltpu.CompilerParams(needs_layout_passes=False,
                                  use_tc_tiling_on_sc=False)


def _half_pad(mat):
    rows = mat.shape[0]
    return jnp.concatenate(
        [mat, jnp.ones((rows, 1), jnp.float32),
         jnp.zeros((rows, PWH - FH - 1), jnp.float32)],
        axis=1)


def _tc1_body(x_ref, w_ref, as_ref, ad_ref, hp_ref, als_ref, ald_ref,
              ms_ref, md_ref):
    i = pl.program_id(0)
    hb = jnp.dot(x_ref[...], w_ref[...], preferred_element_type=jnp.float32)
    for q in range(NQ):
        for j in range(HEADS):
            base = j * HID + q * FH
            hp_ref[q, j] = _half_pad(hb[:, base:base + FH])
    hbr = hb.reshape(BLKP, HEADS, HID)
    als = jnp.sum(hbr * as_ref[...][None], axis=2)
    ald = jnp.sum(hbr * ad_ref[...][None], axis=2)
    als_ref[...] = als.T
    ald_ref[...] = ald.T
    vs = jnp.max(als, axis=0, keepdims=True)
    vd = jnp.max(ald, axis=0, keepdims=True)

    @pl.when(i == 0)
    def _():
        ms_ref[...] = vs
        md_ref[...] = vd

    @pl.when(i > 0)
    def _():
        ms_ref[...] = jnp.maximum(ms_ref[...], vs)
        md_ref[...] = jnp.maximum(md_ref[...], vd)


def _tc1(xp, W1, a_s, a_d):
    return pl.pallas_call(
        _tc1_body,
        grid=(NBP,),
        in_specs=[
            pl.BlockSpec((BLKP, HID), lambda i: (i, 0)),
            pl.BlockSpec((HID, HEADS * HID), lambda i: (0, 0)),
            pl.BlockSpec((HEADS, HID), lambda i: (0, 0)),
            pl.BlockSpec((HEADS, HID), lambda i: (0, 0)),
        ],
        out_specs=[
            pl.BlockSpec((NQ, HEADS, BLKP, PWH), lambda i: (0, 0, i, 0)),
            pl.BlockSpec((HEADS, BLKP), lambda i: (0, i)),
            pl.BlockSpec((HEADS, BLKP), lambda i: (0, i)),
            pl.BlockSpec((1, HEADS), lambda i: (0, 0)),
            pl.BlockSpec((1, HEADS), lambda i: (0, 0)),
        ],
        out_shape=[
            jax.ShapeDtypeStruct((NQ, HEADS, NT, PWH), jnp.float32),
            jax.ShapeDtypeStruct((HEADS, NT), jnp.float32),
            jax.ShapeDtypeStruct((HEADS, NT), jnp.float32),
            jax.ShapeDtypeStruct((1, HEADS), jnp.float32),
            jax.ShapeDtypeStruct((1, HEADS), jnp.float32),
        ],
    )(xp, W1, a_s, a_d)


def _tc3a_body(s_ref, b_ref, g_ref, cs_ref, cq_ref):
    i = pl.program_id(0)
    s = s_ref[...]
    num = jnp.concatenate([s[q, :, :, 0:FH] for q in range(NQ)], axis=2)
    den = s[0, :, :, FH:FH + 1]
    g = num / (den + 1e-16)
    g = jnp.transpose(g, (1, 0, 2)).reshape(BLK, HEADS * HID) + b_ref[...]
    g_ref[...] = g
    cs = jnp.sum(g, axis=0).reshape(HEADS, HID)
    cq = jnp.sum(g * g, axis=0).reshape(HEADS, HID)

    @pl.when(i == 0)
    def _():
        cs_ref[...] = cs
        cq_ref[...] = cq

    @pl.when(i > 0)
    def _():
        cs_ref[...] = cs_ref[...] + cs
        cq_ref[...] = cq_ref[...] + cq


def _tc3a(S1, bias1):
    return pl.pallas_call(
        _tc3a_body,
        grid=(NB,),
        in_specs=[
            pl.BlockSpec((NQ, HEADS, BLK, PWH), lambda i: (0, 0, i, 0)),
            pl.BlockSpec((HEADS * HID,), lambda i: (0,)),
        ],
        out_specs=[
            pl.BlockSpec((BLK, HEADS * HID), lambda i: (i, 0)),
            pl.BlockSpec((HEADS, HID), lambda i: (0, 0)),
            pl.BlockSpec((HEADS, HID), lambda i: (0, 0)),
        ],
        out_shape=[
            jax.ShapeDtypeStruct((N, HEADS * HID), jnp.float32),
            jax.ShapeDtypeStruct((HEADS, HID), jnp.float32),
            jax.ShapeDtypeStruct((HEADS, HID), jnp.float32),
        ],
    )(S1, bias1)


def _tc3b_body(g_ref, cs_ref, cq_ref, gm_ref, bt_ref, w2_ref, as2_ref,
               ad2_ref, hp_ref, als_ref, ald_ref, ms_ref, md_ref):
    i = pl.program_id(0)
    mu = (cs_ref[...] / N).reshape(HEADS * HID)
    var = (cq_ref[...] / N).reshape(HEADS * HID) - mu * mu
    xn = (g_ref[...] - mu) / jnp.sqrt(var + 1e-5) * gm_ref[...] + bt_ref[...]
    h1 = jnp.where(xn > 0, xn, jnp.exp(xn) - 1.0)
    h2 = jnp.dot(h1, w2_ref[...], preferred_element_type=jnp.float32)
    for q in range(NQ):
        hp_ref[q] = _half_pad(h2[:, q * FH:(q + 1) * FH])
    als = jnp.dot(h2, as2_ref[0], preferred_element_type=jnp.float32)
    ald = jnp.dot(h2, ad2_ref[0], preferred_element_type=jnp.float32)
    als_ref[...] = als.reshape(BLK, 1)
    ald_ref[...] = ald.reshape(BLK, 1)
    vs = jnp.full((1, 128), jnp.max(als), jnp.float32)
    vd = jnp.full((1, 128), jnp.max(ald), jnp.float32)

    @pl.when(i == 0)
    def _():
        ms_ref[...] = vs
        md_ref[...] = vd

    @pl.when(i > 0)
    def _():
        ms_ref[...] = jnp.maximum(ms_ref[...], vs)
        md_ref[...] = jnp.maximum(md_ref[...], vd)


def _tc3b(G, CS, CQ, gamma1, beta1, W2, a_s2, a_d2):
    return pl.pallas_call(
        _tc3b_body,
        grid=(NB,),
        in_specs=[
            pl.BlockSpec((BLK, HEADS * HID), lambda i: (i, 0)),
            pl.BlockSpec((HEADS, HID), lambda i: (0, 0)),
            pl.BlockSpec((HEADS, HID), lambda i: (0, 0)),
            pl.BlockSpec((HEADS * HID,), lambda i: (0,)),
            pl.BlockSpec((HEADS * HID,), lambda i: (0,)),
            pl.BlockSpec((HEADS * HID, HID), lambda i: (0, 0)),
            pl.BlockSpec((1, HID), lambda i: (0, 0)),
            pl.BlockSpec((1, HID), lambda i: (0, 0)),
        ],
        out_specs=[
            pl.BlockSpec((NQ, BLK, PWH), lambda i: (0, i, 0)),
            pl.BlockSpec((BLK, 1), lambda i: (i, 0)),
            pl.BlockSpec((BLK, 1), lambda i: (i, 0)),
            pl.BlockSpec((1, 128), lambda i: (0, 0)),
            pl.BlockSpec((1, 128), lambda i: (0, 0)),
        ],
        out_shape=[
            jax.ShapeDtypeStruct((NQ, N, PWH), jnp.float32),
            jax.ShapeDtypeStruct((N, 1), jnp.float32),
            jax.ShapeDtypeStruct((N, 1), jnp.float32),
            jax.ShapeDtypeStruct((1, 128), jnp.float32),
            jax.ShapeDtypeStruct((1, 128), jnp.float32),
        ],
    )(G, CS, CQ, gamma1, beta1, W2, a_s2, a_d2)


def _tc5a_body(s_ref, b_ref, g_ref, cs_ref, cq_ref):
    i = pl.program_id(0)
    s = s_ref[...]
    tot = s[:, 0] + s[:, 1]
    num = jnp.concatenate([tot[q, :, 0:FH] for q in range(NQ)], axis=1)
    den = tot[0, :, FH:FH + 1]
    g = num / (den + 1e-16) + b_ref[...]
    g_ref[...] = g
    cs = jnp.sum(g, axis=0, keepdims=True)
    cq = jnp.sum(g * g, axis=0, keepdims=True)

    @pl.when(i == 0)
    def _():
        cs_ref[...] = cs
        cq_ref[...] = cq

    @pl.when(i > 0)
    def _():
        cs_ref[...] = cs_ref[...] + cs
        cq_ref[...] = cq_ref[...] + cq


def _tc5a(S2P, bias2):
    return pl.pallas_call(
        _tc5a_body,
        grid=(NB,),
        in_specs=[
            pl.BlockSpec((NQ, 2, BLK, PWH), lambda i: (0, 0, i, 0)),
            pl.BlockSpec((HID,), lambda i: (0,)),
        ],
        out_specs=[
            pl.BlockSpec((BLK, HID), lambda i: (i, 0)),
            pl.BlockSpec((1, HID), lambda i: (0, 0)),
            pl.BlockSpec((1, HID), lambda i: (0, 0)),
        ],
        out_shape=[
            jax.ShapeDtypeStruct((N, HID), jnp.float32),
            jax.ShapeDtypeStruct((1, HID), jnp.float32),
            jax.ShapeDtypeStruct((1, HID), jnp.float32),
        ],
    )(S2P, bias2)


def _tc5b_body(g_ref, cs_ref, cq_ref, gm_ref, bt_ref, w1_ref, b1_ref,
               w2_ref, b2_ref, o_ref):
    mu = cs_ref[0] / N
    var = cq_ref[0] / N - mu * mu
    xn = (g_ref[...] - mu) / jnp.sqrt(var + 1e-5) * gm_ref[...] + bt_ref[...]
    h = jnp.where(xn > 0, xn, jnp.exp(xn) - 1.0)
    z = jnp.maximum(
        jnp.dot(h, w1_ref[...], preferred_element_type=jnp.float32) + b1_ref[...], 0.0)
    o_ref[...] = jnp.dot(z, w2_ref[...], preferred_element_type=jnp.float32) + b2_ref[...]


def _tc5b(G2, CS2, CQ2, gamma2, beta2, cls_W1, cls_b1, w2p, b2p):
    return pl.pallas_call(
        _tc5b_body,
        grid=(NB,),
        in_specs=[
            pl.BlockSpec((BLK, HID), lambda i: (i, 0)),
            pl.BlockSpec((1, HID), lambda i: (0, 0)),
            pl.BlockSpec((1, HID), lambda i: (0, 0)),
            pl.BlockSpec((HID,), lambda i: (0,)),
            pl.BlockSpec((HID,), lambda i: (0,)),
            pl.BlockSpec((HID, 64), lambda i: (0, 0)),
            pl.BlockSpec((64,), lambda i: (0,)),
            pl.BlockSpec((64, 128), lambda i: (0, 0)),
            pl.BlockSpec((128,), lambda i: (0,)),
        ],
        out_specs=pl.BlockSpec((BLK, 128), lambda i: (i, 0)),
        out_shape=jax.ShapeDtypeStruct((N, 128), jnp.float32),
    )(G2, CS2, CQ2, gamma2, beta2, cls_W1, cls_b1, w2p, b2p)


def _edge_chunk(hp, acc, src_v, dstc_v, als_v, ald_v, gidx_v, exc_v, rows_v,
                g, hoff, m):
    for v in range(KCH // 16):
        sl = pl.ds(v * 16, 16)
        s16 = src_v[pl.ds(g * KCH + v * 16, 16)]
        d16 = dstc_v[g, sl]
        a1 = plsc.load_gather(als_v, [s16])
        a2 = plsc.load_gather(ald_v, [d16])
        e = a1 + a2
        e = jnp.maximum(e, e * 0.2)
        exc_v[sl] = jnp.exp(e - m)
        gidx_v[sl] = s16 + hoff
    pltpu.sync_copy(hp.at[gidx_v], rows_v)
    for v in range(KCH // 16):
        ex16 = exc_v[pl.ds(v * 16, 16)]
        r0 = v * 16
        for rr in range(16):
            exs = ex16[rr]
            for j in range(PWH // 16):
                sl2 = pl.ds(j * 16, 16)
                rows_v[r0 + rr, sl2] = rows_v[r0 + rr, sl2] * exs
    pltpu.sync_copy(rows_v, acc.at[dstc_v.at[g]], add=True)


def _zero_acc(rows_v, acc, rbase):
    def zcopy(k, _):
        pltpu.sync_copy(rows_v, acc.at[pl.ds(rbase + k * KCH, KCH), :])
        return _

    lax.fori_loop(0, RZ // KCH, zcopy, 0)
    rem = RZ - (RZ // KCH) * KCH
    if rem:
        pltpu.sync_copy(rows_v.at[pl.ds(0, rem), :],
                        acc.at[pl.ds(rbase + (RZ // KCH) * KCH, rem), :])


def _zero_rows(rows_v):
    def zrow(r, _):
        for j in range(PWH // 16):
            rows_v[r, pl.ds(j * 16, 16)] = jnp.zeros((16,), jnp.float32)
        return _

    lax.fori_loop(0, KCH, zrow, 0)


def _sc_l1_body(hp, als, ald, srcr, dst3, m32, s1,
                als_v, ald_v, src_v, dstc_v, gidx_v, exc_v, rows_v, m_v, acc):
    c = lax.axis_index("c")
    s = lax.axis_index("s")
    rbase = s * RZ
    pltpu.sync_copy(m32, m_v)
    pltpu.sync_copy(dst3.at[s], dstc_v)
    pltpu.sync_copy(srcr.at[pl.ds(s * EPT1, EPT1)], src_v)

    for hh in range(HEADS // 2):
        hI = c * (HEADS // 2) + hh
        pltpu.sync_copy(als.at[hI], als_v)
        pltpu.sync_copy(ald.at[hI], ald_v)
        m = m_v[pl.ds(hI, 16)][0]

        for q in range(NQ):
            _zero_rows(rows_v)

            @pl.when(s < RTILES)
            def _():
                _zero_acc(rows_v, acc, rbase)

            plsc.subcore_barrier()
            hoff = (q * HEADS + hI) * NT

            def chunk_body(g, _):
                _edge_chunk(hp, acc, src_v, dstc_v, als_v, ald_v, gidx_v,
                            exc_v, rows_v, g, hoff, m)
                return _

            lax.fori_loop(0, NCH1, chunk_body, 0)
            plsc.subcore_barrier()

            @pl.when(s < RTILES)
            def _():
                pltpu.sync_copy(acc.at[pl.ds(rbase, RZ), :],
                                s1.at[q, hI, pl.ds(rbase, RZ), :])

            plsc.subcore_barrier()


def _sc_l1(H1flat, ALS, ALD, src, dst3, M32):
    f = pl.kernel(
        _sc_l1_body,
        out_type=jax.ShapeDtypeStruct((NQ, HEADS, NACC, PWH), jnp.float32),
        mesh=plsc.VectorSubcoreMesh(core_axis_name="c", subcore_axis_name="s"),
        compiler_params=_SC_PARAMS,
        scratch_types=[
            pltpu.VMEM((NT,), jnp.float32),
            pltpu.VMEM((NT,), jnp.float32),
            pltpu.VMEM((EPT1,), jnp.int32),
            pltpu.VMEM((NCH1, KCH), jnp.int32),
            pltpu.VMEM((KCH,), jnp.int32),
            pltpu.VMEM((KCH,), jnp.float32),
            pltpu.VMEM((KCH, PWH), jnp.float32),
            pltpu.VMEM((32,), jnp.float32),
            pltpu.VMEM_SHARED((NACC, PWH), jnp.float32),
        ],
    )
    return f(H1flat, ALS, ALD, src, dst3, M32)


def _sc_l2_body(hp, als, ald, srcr, dst3, m32, s2p,
                als_v, ald_v, src_v, dstc_v, gidx_v, exc_v, rows_v, m_v, acc):
    c = lax.axis_index("c")
    s = lax.axis_index("s")
    w = c * 16 + s
    rbase = s * RZ
    pltpu.sync_copy(m32, m_v)
    pltpu.sync_copy(dst3.at[w], dstc_v)
    pltpu.sync_copy(srcr.at[pl.ds(w * EPT2, EPT2)], src_v)
    pltpu.sync_copy(als, als_v)
    pltpu.sync_copy(ald, ald_v)
    m = m_v[pl.ds(0, 16)][0]

    for q in range(NQ):
        _zero_rows(rows_v)

        @pl.when(s < RTILES)
        def _():
            _zero_acc(rows_v, acc, rbase)

        plsc.subcore_barrier()
        hoff = q * N

        def chunk_body(g, _):
            _edge_chunk(hp, acc, src_v, dstc_v, als_v, ald_v, gidx_v,
                        exc_v, rows_v, g, hoff, m)
            return _

        lax.fori_loop(0, NCH2, chunk_body, 0)
        plsc.subcore_barrier()

        @pl.when(s < RTILES)
        def _():
            pltpu.sync_copy(acc.at[pl.ds(rbase, RZ), :],
                            s2p.at[q, c, pl.ds(rbase, RZ), :])

        plsc.subcore_barrier()


def _sc_l2(H2flat, ALS2, ALD2, src, dst3, M32):
    f = pl.kernel(
        _sc_l2_body,
        out_type=jax.ShapeDtypeStruct((NQ, 2, NACC, PWH), jnp.float32),
        mesh=plsc.VectorSubcoreMesh(core_axis_name="c", subcore_axis_name="s"),
        compiler_params=_SC_PARAMS,
        scratch_types=[
            pltpu.VMEM((N,), jnp.float32),
            pltpu.VMEM((N,), jnp.float32),
            pltpu.VMEM((EPT2,), jnp.int32),
            pltpu.VMEM((NCH2, KCH), jnp.int32),
            pltpu.VMEM((KCH,), jnp.int32),
            pltpu.VMEM((KCH,), jnp.float32),
            pltpu.VMEM((KCH, PWH), jnp.float32),
            pltpu.VMEM((32,), jnp.float32),
            pltpu.VMEM_SHARED((NACC, PWH), jnp.float32),
        ],
    )
    return f(H2flat, ALS2, ALD2, src, dst3, M32)


def _leaky(x):
    return jnp.where(x > 0, x, 0.2 * x)


def kernel(x, edge_index, W1, att_src1, att_dst1, bias1, gamma1, beta1,
           W2, att_src2, att_dst2, bias2, gamma2, beta2,
           cls_W1, cls_b1, cls_W2, cls_b2):
    src = edge_index[0]
    dst = edge_index[1]
    dst31 = dst.reshape(16, NCH1, KCH)
    dst32 = dst.reshape(32, NCH2, KCH)

    xp = jnp.pad(x, ((0, NT - N), (0, 0)))
    H1p, ALS, ALD, MS, MD = _tc1(xp, W1, att_src1, att_dst1)
    M1 = _leaky(MS[0] + MD[0])
    M32 = jnp.concatenate([M1, jnp.zeros((24,), jnp.float32)])

    S1 = _sc_l1(H1p.reshape(NQ * HEADS * NT, PWH), ALS, ALD,
                src, dst31, M32)

    G, CS, CQ = _tc3a(S1, bias1)
    H2p, ALS2, ALD2, MS2, MD2 = _tc3b(G, CS, CQ, gamma1, beta1, W2,
                                      att_src2, att_dst2)
    M2 = _leaky(MS2[0, 0] + MD2[0, 0])
    M232 = jnp.full((32,), M2, jnp.float32)

    S2P = _sc_l2(H2p.reshape(NQ * N, PWH), ALS2.reshape(N), ALD2.reshape(N),
                 src, dst32, M232)

    G2, CS2, CQ2 = _tc5a(S2P, bias2)
    w2p = jnp.pad(cls_W2, ((0, 0), (0, 126)))
    b2p = jnp.pad(cls_b2, (0, 126))
    out = _tc5b(G2, CS2, CQ2, gamma2, beta2, cls_W1, cls_b1, w2p, b2p)
    return out[:, :2]


# speedup vs baseline: 22.2349x; 1.7110x over previous
import jax
import jax.numpy as jnp
from jax import lax
from jax.experimental import pallas as pl
from jax.experimental.pallas import tpu as pltpu
from jax.experimental.pallas import tpu_sc as plsc

N = 10000
E = 320000
HEADS = 8
HID = 128
FH = 64
NQ = 2
PWH = 80
BLK = 400
NB = N // BLK
KCH = 80
EPT1 = E // 16
NCH1 = EPT1 // KCH
EPT2 = E // 32
NCH2 = EPT2 // KCH
NACC = 10000
RTILES = 10
RZ = NACC // RTILES
NT = 10240
BLKP = 512
NBP = NT // BLKP

_SC_PARAMS = pltpu.CompilerParams(needs_layout_passes=False,
                                  use_tc_tiling_on_sc=False)


def _half_pad(mat):
    rows = mat.shape[0]
    return jnp.concatenate(
        [mat, jnp.ones((rows, 1), jnp.float32),
         jnp.zeros((rows, PWH - FH - 1), jnp.float32)],
        axis=1)


def _tc1_body(x_ref, w_ref, as_ref, ad_ref, hp_ref, als_ref, ald_ref,
              ms_ref, md_ref):
    i = pl.program_id(0)
    hb = jnp.dot(x_ref[...], w_ref[...], preferred_element_type=jnp.float32)
    for q in range(NQ):
        for j in range(HEADS):
            base = j * HID + q * FH
            hp_ref[q, j] = _half_pad(hb[:, base:base + FH])
    hbr = hb.reshape(BLKP, HEADS, HID)
    als = jnp.sum(hbr * as_ref[...][None], axis=2)
    ald = jnp.sum(hbr * ad_ref[...][None], axis=2)
    als_ref[...] = als.T
    ald_ref[...] = ald.T
    vs = jnp.max(als, axis=0, keepdims=True)
    vd = jnp.max(ald, axis=0, keepdims=True)

    @pl.when(i == 0)
    def _():
        ms_ref[...] = vs
        md_ref[...] = vd

    @pl.when(i > 0)
    def _():
        ms_ref[...] = jnp.maximum(ms_ref[...], vs)
        md_ref[...] = jnp.maximum(md_ref[...], vd)


def _tc1(xp, W1, a_s, a_d):
    return pl.pallas_call(
        _tc1_body,
        grid=(NBP,),
        in_specs=[
            pl.BlockSpec((BLKP, HID), lambda i: (i, 0)),
            pl.BlockSpec((HID, HEADS * HID), lambda i: (0, 0)),
            pl.BlockSpec((HEADS, HID), lambda i: (0, 0)),
            pl.BlockSpec((HEADS, HID), lambda i: (0, 0)),
        ],
        out_specs=[
            pl.BlockSpec((NQ, HEADS, BLKP, PWH), lambda i: (0, 0, i, 0)),
            pl.BlockSpec((HEADS, BLKP), lambda i: (0, i)),
            pl.BlockSpec((HEADS, BLKP), lambda i: (0, i)),
            pl.BlockSpec((1, HEADS), lambda i: (0, 0)),
            pl.BlockSpec((1, HEADS), lambda i: (0, 0)),
        ],
        out_shape=[
            jax.ShapeDtypeStruct((NQ, HEADS, NT, PWH), jnp.float32),
            jax.ShapeDtypeStruct((HEADS, NT), jnp.float32),
            jax.ShapeDtypeStruct((HEADS, NT), jnp.float32),
            jax.ShapeDtypeStruct((1, HEADS), jnp.float32),
            jax.ShapeDtypeStruct((1, HEADS), jnp.float32),
        ],
    )(xp, W1, a_s, a_d)


def _tc3a_body(s_ref, b_ref, g_ref, cs_ref, cq_ref):
    i = pl.program_id(0)
    s = s_ref[...]
    num = jnp.concatenate([s[q, :, :, 0:FH] for q in range(NQ)], axis=2)
    den = s[0, :, :, FH:FH + 1]
    g = num / (den + 1e-16)
    g = jnp.transpose(g, (1, 0, 2)).reshape(BLK, HEADS * HID) + b_ref[...]
    g_ref[...] = g
    cs = jnp.sum(g, axis=0).reshape(HEADS, HID)
    cq = jnp.sum(g * g, axis=0).reshape(HEADS, HID)

    @pl.when(i == 0)
    def _():
        cs_ref[...] = cs
        cq_ref[...] = cq

    @pl.when(i > 0)
    def _():
        cs_ref[...] = cs_ref[...] + cs
        cq_ref[...] = cq_ref[...] + cq


def _tc3a(S1, bias1):
    return pl.pallas_call(
        _tc3a_body,
        grid=(NB,),
        in_specs=[
            pl.BlockSpec((NQ, HEADS, BLK, PWH), lambda i: (0, 0, i, 0)),
            pl.BlockSpec((HEADS * HID,), lambda i: (0,)),
        ],
        out_specs=[
            pl.BlockSpec((BLK, HEADS * HID), lambda i: (i, 0)),
            pl.BlockSpec((HEADS, HID), lambda i: (0, 0)),
            pl.BlockSpec((HEADS, HID), lambda i: (0, 0)),
        ],
        out_shape=[
            jax.ShapeDtypeStruct((N, HEADS * HID), jnp.float32),
            jax.ShapeDtypeStruct((HEADS, HID), jnp.float32),
            jax.ShapeDtypeStruct((HEADS, HID), jnp.float32),
        ],
    )(S1, bias1)


def _tc3b_body(g_ref, cs_ref, cq_ref, gm_ref, bt_ref, w2_ref, as2_ref,
               ad2_ref, hp_ref, als_ref, ald_ref, ms_ref, md_ref):
    i = pl.program_id(0)
    mu = (cs_ref[...] / N).reshape(HEADS * HID)
    var = (cq_ref[...] / N).reshape(HEADS * HID) - mu * mu
    xn = (g_ref[...] - mu) / jnp.sqrt(var + 1e-5) * gm_ref[...] + bt_ref[...]
    h1 = jnp.where(xn > 0, xn, jnp.exp(xn) - 1.0)
    h2 = jnp.dot(h1, w2_ref[...], preferred_element_type=jnp.float32)
    for q in range(NQ):
        hp_ref[q] = _half_pad(h2[:, q * FH:(q + 1) * FH])
    als = jnp.dot(h2, as2_ref[0], preferred_element_type=jnp.float32)
    ald = jnp.dot(h2, ad2_ref[0], preferred_element_type=jnp.float32)
    als_ref[...] = als.reshape(BLK, 1)
    ald_ref[...] = ald.reshape(BLK, 1)
    vs = jnp.full((1, 128), jnp.max(als), jnp.float32)
    vd = jnp.full((1, 128), jnp.max(ald), jnp.float32)

    @pl.when(i == 0)
    def _():
        ms_ref[...] = vs
        md_ref[...] = vd

    @pl.when(i > 0)
    def _():
        ms_ref[...] = jnp.maximum(ms_ref[...], vs)
        md_ref[...] = jnp.maximum(md_ref[...], vd)


def _tc3b(G, CS, CQ, gamma1, beta1, W2, a_s2, a_d2):
    return pl.pallas_call(
        _tc3b_body,
        grid=(NB,),
        in_specs=[
            pl.BlockSpec((BLK, HEADS * HID), lambda i: (i, 0)),
            pl.BlockSpec((HEADS, HID), lambda i: (0, 0)),
            pl.BlockSpec((HEADS, HID), lambda i: (0, 0)),
            pl.BlockSpec((HEADS * HID,), lambda i: (0,)),
            pl.BlockSpec((HEADS * HID,), lambda i: (0,)),
            pl.BlockSpec((HEADS * HID, HID), lambda i: (0, 0)),
            pl.BlockSpec((1, HID), lambda i: (0, 0)),
            pl.BlockSpec((1, HID), lambda i: (0, 0)),
        ],
        out_specs=[
            pl.BlockSpec((NQ, BLK, PWH), lambda i: (0, i, 0)),
            pl.BlockSpec((BLK, 1), lambda i: (i, 0)),
            pl.BlockSpec((BLK, 1), lambda i: (i, 0)),
            pl.BlockSpec((1, 128), lambda i: (0, 0)),
            pl.BlockSpec((1, 128), lambda i: (0, 0)),
        ],
        out_shape=[
            jax.ShapeDtypeStruct((NQ, N, PWH), jnp.float32),
            jax.ShapeDtypeStruct((N, 1), jnp.float32),
            jax.ShapeDtypeStruct((N, 1), jnp.float32),
            jax.ShapeDtypeStruct((1, 128), jnp.float32),
            jax.ShapeDtypeStruct((1, 128), jnp.float32),
        ],
    )(G, CS, CQ, gamma1, beta1, W2, a_s2, a_d2)


def _tc5a_body(s_ref, b_ref, g_ref, cs_ref, cq_ref):
    i = pl.program_id(0)
    s = s_ref[...]
    tot = s[:, 0] + s[:, 1]
    num = jnp.concatenate([tot[q, :, 0:FH] for q in range(NQ)], axis=1)
    den = tot[0, :, FH:FH + 1]
    g = num / (den + 1e-16) + b_ref[...]
    g_ref[...] = g
    cs = jnp.sum(g, axis=0, keepdims=True)
    cq = jnp.sum(g * g, axis=0, keepdims=True)

    @pl.when(i == 0)
    def _():
        cs_ref[...] = cs
        cq_ref[...] = cq

    @pl.when(i > 0)
    def _():
        cs_ref[...] = cs_ref[...] + cs
        cq_ref[...] = cq_ref[...] + cq


def _tc5a(S2P, bias2):
    return pl.pallas_call(
        _tc5a_body,
        grid=(NB,),
        in_specs=[
            pl.BlockSpec((NQ, 2, BLK, PWH), lambda i: (0, 0, i, 0)),
            pl.BlockSpec((HID,), lambda i: (0,)),
        ],
        out_specs=[
            pl.BlockSpec((BLK, HID), lambda i: (i, 0)),
            pl.BlockSpec((1, HID), lambda i: (0, 0)),
            pl.BlockSpec((1, HID), lambda i: (0, 0)),
        ],
        out_shape=[
            jax.ShapeDtypeStruct((N, HID), jnp.float32),
            jax.ShapeDtypeStruct((1, HID), jnp.float32),
            jax.ShapeDtypeStruct((1, HID), jnp.float32),
        ],
    )(S2P, bias2)


def _tc5b_body(g_ref, cs_ref, cq_ref, gm_ref, bt_ref, w1_ref, b1_ref,
               w2_ref, b2_ref, o_ref):
    mu = cs_ref[0] / N
    var = cq_ref[0] / N - mu * mu
    xn = (g_ref[...] - mu) / jnp.sqrt(var + 1e-5) * gm_ref[...] + bt_ref[...]
    h = jnp.where(xn > 0, xn, jnp.exp(xn) - 1.0)
    z = jnp.maximum(
        jnp.dot(h, w1_ref[...], preferred_element_type=jnp.float32) + b1_ref[...], 0.0)
    o_ref[...] = jnp.dot(z, w2_ref[...], preferred_element_type=jnp.float32) + b2_ref[...]


def _tc5b(G2, CS2, CQ2, gamma2, beta2, cls_W1, cls_b1, w2p, b2p):
    return pl.pallas_call(
        _tc5b_body,
        grid=(NB,),
        in_specs=[
            pl.BlockSpec((BLK, HID), lambda i: (i, 0)),
            pl.BlockSpec((1, HID), lambda i: (0, 0)),
            pl.BlockSpec((1, HID), lambda i: (0, 0)),
            pl.BlockSpec((HID,), lambda i: (0,)),
            pl.BlockSpec((HID,), lambda i: (0,)),
            pl.BlockSpec((HID, 64), lambda i: (0, 0)),
            pl.BlockSpec((64,), lambda i: (0,)),
            pl.BlockSpec((64, 128), lambda i: (0, 0)),
            pl.BlockSpec((128,), lambda i: (0,)),
        ],
        out_specs=pl.BlockSpec((BLK, 128), lambda i: (i, 0)),
        out_shape=jax.ShapeDtypeStruct((N, 128), jnp.float32),
    )(G2, CS2, CQ2, gamma2, beta2, cls_W1, cls_b1, w2p, b2p)


def _prep_chunk(src_v, dstc_v, als_v, ald_v, gidx_v, exc_v, b, g, hoff, m):
    for v in range(KCH // 16):
        sl = pl.ds(v * 16, 16)
        s16 = src_v[pl.ds(g * KCH + v * 16, 16)]
        d16 = dstc_v[g, sl]
        a1 = plsc.load_gather(als_v, [s16])
        a2 = plsc.load_gather(ald_v, [d16])
        e = a1 + a2
        e = jnp.maximum(e, e * 0.2)
        exc_v[b, sl] = jnp.exp(e - m)
        gidx_v[b, sl] = s16 + hoff


def _scale_chunk(exc_v, rows_v, b):
    for v in range(KCH // 16):
        ex16 = exc_v[b, pl.ds(v * 16, 16)]
        r0 = v * 16
        for rr in range(16):
            exs = ex16[rr]
            for j in range(PWH // 16):
                sl2 = pl.ds(j * 16, 16)
                rows_v[b, r0 + rr, sl2] = rows_v[b, r0 + rr, sl2] * exs


def _edge_loop(hp, acc, src_v, dstc_v, als_v, ald_v, gidx_v, exc_v, rows_v,
               gsem, ssem, nch, hoff, m):
    def g_copy(b, g):
        return pltpu.make_async_copy(hp.at[gidx_v.at[b]], rows_v.at[b],
                                     gsem.at[b])

    def s_copy(b, g):
        return pltpu.make_async_copy(rows_v.at[b], acc.at[dstc_v.at[g]],
                                     ssem.at[b])

    _prep_chunk(src_v, dstc_v, als_v, ald_v, gidx_v, exc_v, 0, 0, hoff, m)
    g_copy(0, 0).start()

    def chunk_body(g, carry):
        b = lax.rem(g, 2)
        nb = 1 - b

        @pl.when(g >= 1)
        def _w1():
            s_copy(nb, g - 1).wait()

        @pl.when(g + 1 < nch)
        def _w2():
            _prep_chunk(src_v, dstc_v, als_v, ald_v, gidx_v, exc_v, nb,
                        g + 1, hoff, m)
            g_copy(nb, g + 1).start()

        g_copy(b, g).wait()
        _scale_chunk(exc_v, rows_v, b)
        s_copy(b, g).start(add=True)
        return carry

    lax.fori_loop(0, nch, chunk_body, 0)
    s_copy((nch - 1) % 2, nch - 1).wait()


def _zero_acc(rows_v, acc, rbase):
    def zcopy(k, _):
        pltpu.sync_copy(rows_v.at[0], acc.at[pl.ds(rbase + k * KCH, KCH), :])
        return _

    lax.fori_loop(0, RZ // KCH, zcopy, 0)
    rem = RZ - (RZ // KCH) * KCH
    if rem:
        pltpu.sync_copy(rows_v.at[0, pl.ds(0, rem), :],
                        acc.at[pl.ds(rbase + (RZ // KCH) * KCH, rem), :])


def _zero_rows(rows_v):
    def zrow(r, _):
        for j in range(PWH // 16):
            rows_v[0, r, pl.ds(j * 16, 16)] = jnp.zeros((16,), jnp.float32)
        return _

    lax.fori_loop(0, KCH, zrow, 0)


def _sc_l1_body(hp, als, ald, srcr, dst3, m32, s1,
                als_v, ald_v, src_v, dstc_v, gidx_v, exc_v, rows_v, m_v,
                gsem, ssem, acc):
    c = lax.axis_index("c")
    s = lax.axis_index("s")
    rbase = s * RZ
    pltpu.sync_copy(m32, m_v)
    pltpu.sync_copy(dst3.at[s], dstc_v)
    pltpu.sync_copy(srcr.at[pl.ds(s * EPT1, EPT1)], src_v)

    for hh in range(HEADS // 2):
        hI = c * (HEADS // 2) + hh
        pltpu.sync_copy(als.at[hI], als_v)
        pltpu.sync_copy(ald.at[hI], ald_v)
        m = m_v[pl.ds(hI, 16)][0]

        for q in range(NQ):
            _zero_rows(rows_v)

            @pl.when(s < RTILES)
            def _():
                _zero_acc(rows_v, acc, rbase)

            plsc.subcore_barrier()
            hoff = (q * HEADS + hI) * NT
            _edge_loop(hp, acc, src_v, dstc_v, als_v, ald_v, gidx_v, exc_v,
                       rows_v, gsem, ssem, NCH1, hoff, m)
            plsc.subcore_barrier()

            @pl.when(s < RTILES)
            def _():
                pltpu.sync_copy(acc.at[pl.ds(rbase, RZ), :],
                                s1.at[q, hI, pl.ds(rbase, RZ), :])

            plsc.subcore_barrier()


def _sc_l1(H1flat, ALS, ALD, src, dst3, M32):
    f = pl.kernel(
        _sc_l1_body,
        out_type=jax.ShapeDtypeStruct((NQ, HEADS, NACC, PWH), jnp.float32),
        mesh=plsc.VectorSubcoreMesh(core_axis_name="c", subcore_axis_name="s"),
        compiler_params=_SC_PARAMS,
        scratch_types=[
            pltpu.VMEM((NT,), jnp.float32),
            pltpu.VMEM((NT,), jnp.float32),
            pltpu.VMEM((EPT1,), jnp.int32),
            pltpu.VMEM((NCH1, KCH), jnp.int32),
            pltpu.VMEM((2, KCH), jnp.int32),
            pltpu.VMEM((2, KCH), jnp.float32),
            pltpu.VMEM((2, KCH, PWH), jnp.float32),
            pltpu.VMEM((32,), jnp.float32),
            pltpu.SemaphoreType.DMA((2,)),
            pltpu.SemaphoreType.DMA((2,)),
            pltpu.VMEM_SHARED((NACC, PWH), jnp.float32),
        ],
    )
    return f(H1flat, ALS, ALD, src, dst3, M32)


def _sc_l2_body(hp, als, ald, srcr, dst3, m32, s2p,
                als_v, ald_v, src_v, dstc_v, gidx_v, exc_v, rows_v, m_v,
                gsem, ssem, acc):
    c = lax.axis_index("c")
    s = lax.axis_index("s")
    w = c * 16 + s
    rbase = s * RZ
    pltpu.sync_copy(m32, m_v)
    pltpu.sync_copy(dst3.at[w], dstc_v)
    pltpu.sync_copy(srcr.at[pl.ds(w * EPT2, EPT2)], src_v)
    pltpu.sync_copy(als, als_v)
    pltpu.sync_copy(ald, ald_v)
    m = m_v[pl.ds(0, 16)][0]

    for q in range(NQ):
        _zero_rows(rows_v)

        @pl.when(s < RTILES)
        def _():
            _zero_acc(rows_v, acc, rbase)

        plsc.subcore_barrier()
        hoff = q * N
        _edge_loop(hp, acc, src_v, dstc_v, als_v, ald_v, gidx_v, exc_v,
                   rows_v, gsem, ssem, NCH2, hoff, m)
        plsc.subcore_barrier()

        @pl.when(s < RTILES)
        def _():
            pltpu.sync_copy(acc.at[pl.ds(rbase, RZ), :],
                            s2p.at[q, c, pl.ds(rbase, RZ), :])

        plsc.subcore_barrier()


def _sc_l2(H2flat, ALS2, ALD2, src, dst3, M32):
    f = pl.kernel(
        _sc_l2_body,
        out_type=jax.ShapeDtypeStruct((NQ, 2, NACC, PWH), jnp.float32),
        mesh=plsc.VectorSubcoreMesh(core_axis_name="c", subcore_axis_name="s"),
        compiler_params=_SC_PARAMS,
        scratch_types=[
            pltpu.VMEM((N,), jnp.float32),
            pltpu.VMEM((N,), jnp.float32),
            pltpu.VMEM((EPT2,), jnp.int32),
            pltpu.VMEM((NCH2, KCH), jnp.int32),
            pltpu.VMEM((2, KCH), jnp.int32),
            pltpu.VMEM((2, KCH), jnp.float32),
            pltpu.VMEM((2, KCH, PWH), jnp.float32),
            pltpu.VMEM((32,), jnp.float32),
            pltpu.SemaphoreType.DMA((2,)),
            pltpu.SemaphoreType.DMA((2,)),
            pltpu.VMEM_SHARED((NACC, PWH), jnp.float32),
        ],
    )
    return f(H2flat, ALS2, ALD2, src, dst3, M32)


def _leaky(x):
    return jnp.where(x > 0, x, 0.2 * x)


def kernel(x, edge_index, W1, att_src1, att_dst1, bias1, gamma1, beta1,
           W2, att_src2, att_dst2, bias2, gamma2, beta2,
           cls_W1, cls_b1, cls_W2, cls_b2):
    src = edge_index[0]
    dst = edge_index[1]
    dst31 = dst.reshape(16, NCH1, KCH)
    dst32 = dst.reshape(32, NCH2, KCH)

    xp = jnp.pad(x, ((0, NT - N), (0, 0)))
    H1p, ALS, ALD, MS, MD = _tc1(xp, W1, att_src1, att_dst1)
    M1 = _leaky(MS[0] + MD[0])
    M32 = jnp.concatenate([M1, jnp.zeros((24,), jnp.float32)])

    S1 = _sc_l1(H1p.reshape(NQ * HEADS * NT, PWH), ALS, ALD,
                src, dst31, M32)

    G, CS, CQ = _tc3a(S1, bias1)
    H2p, ALS2, ALD2, MS2, MD2 = _tc3b(G, CS, CQ, gamma1, beta1, W2,
                                      att_src2, att_dst2)
    M2 = _leaky(MS2[0, 0] + MD2[0, 0])
    M232 = jnp.full((32,), M2, jnp.float32)

    S2P = _sc_l2(H2p.reshape(NQ * N, PWH), ALS2.reshape(N), ALD2.reshape(N),
                 src, dst32, M232)

    G2, CS2, CQ2 = _tc5a(S2P, bias2)
    w2p = jnp.pad(cls_W2, ((0, 0), (0, 126)))
    b2p = jnp.pad(cls_b2, (0, 126))
    out = _tc5b(G2, CS2, CQ2, gamma2, beta2, cls_W1, cls_b1, w2p, b2p)
    return out[:, :2]


# speedup vs baseline: 22.2406x; 1.0003x over previous
import jax
import jax.numpy as jnp
from jax import lax
from jax.experimental import pallas as pl
from jax.experimental.pallas import tpu as pltpu
from jax.experimental.pallas import tpu_sc as plsc

N = 10000
E = 320000
HEADS = 8
HID = 128
FH = 64
NQ = 2
PWH = 80
BLK = 400
NB = N // BLK
KCH = 80
EPT1 = E // 16
NCH1 = EPT1 // KCH
EPT2 = E // 32
NCH2 = EPT2 // KCH
NACC = 10000
RTILES = 10
RZ = NACC // RTILES
NT = 10240
BLKP = 512
NBP = NT // BLKP

_SC_PARAMS = pltpu.CompilerParams(needs_layout_passes=False,
                                  use_tc_tiling_on_sc=False)


def _half_pad(mat):
    rows = mat.shape[0]
    return jnp.concatenate(
        [mat, jnp.ones((rows, 1), jnp.float32),
         jnp.zeros((rows, PWH - FH - 1), jnp.float32)],
        axis=1)


def _tc1_body(x_ref, w_ref, as_ref, ad_ref, hp_ref, als_ref, ald_ref,
              ms_ref, md_ref):
    i = pl.program_id(0)
    hb = jnp.dot(x_ref[...], w_ref[...], preferred_element_type=jnp.float32)
    for q in range(NQ):
        for j in range(HEADS):
            base = j * HID + q * FH
            hp_ref[q, j] = _half_pad(hb[:, base:base + FH])
    hbr = hb.reshape(BLKP, HEADS, HID)
    als = jnp.sum(hbr * as_ref[...][None], axis=2)
    ald = jnp.sum(hbr * ad_ref[...][None], axis=2)
    als_ref[...] = als.T
    ald_ref[...] = ald.T
    vs = jnp.max(als, axis=0, keepdims=True)
    vd = jnp.max(ald, axis=0, keepdims=True)

    @pl.when(i == 0)
    def _():
        ms_ref[...] = vs
        md_ref[...] = vd

    @pl.when(i > 0)
    def _():
        ms_ref[...] = jnp.maximum(ms_ref[...], vs)
        md_ref[...] = jnp.maximum(md_ref[...], vd)


def _tc1(xp, W1, a_s, a_d):
    return pl.pallas_call(
        _tc1_body,
        grid=(NBP,),
        in_specs=[
            pl.BlockSpec((BLKP, HID), lambda i: (i, 0)),
            pl.BlockSpec((HID, HEADS * HID), lambda i: (0, 0)),
            pl.BlockSpec((HEADS, HID), lambda i: (0, 0)),
            pl.BlockSpec((HEADS, HID), lambda i: (0, 0)),
        ],
        out_specs=[
            pl.BlockSpec((NQ, HEADS, BLKP, PWH), lambda i: (0, 0, i, 0)),
            pl.BlockSpec((HEADS, BLKP), lambda i: (0, i)),
            pl.BlockSpec((HEADS, BLKP), lambda i: (0, i)),
            pl.BlockSpec((1, HEADS), lambda i: (0, 0)),
            pl.BlockSpec((1, HEADS), lambda i: (0, 0)),
        ],
        out_shape=[
            jax.ShapeDtypeStruct((NQ, HEADS, NT, PWH), jnp.float32),
            jax.ShapeDtypeStruct((HEADS, NT), jnp.float32),
            jax.ShapeDtypeStruct((HEADS, NT), jnp.float32),
            jax.ShapeDtypeStruct((1, HEADS), jnp.float32),
            jax.ShapeDtypeStruct((1, HEADS), jnp.float32),
        ],
    )(xp, W1, a_s, a_d)


def _tc3a_body(s_ref, b_ref, g_ref, cs_ref, cq_ref):
    i = pl.program_id(0)
    s = s_ref[...]
    num = jnp.concatenate([s[q, :, :, 0:FH] for q in range(NQ)], axis=2)
    den = s[0, :, :, FH:FH + 1]
    g = num / (den + 1e-16)
    g = jnp.transpose(g, (1, 0, 2)).reshape(BLK, HEADS * HID) + b_ref[...]
    g_ref[...] = g
    cs = jnp.sum(g, axis=0).reshape(HEADS, HID)
    cq = jnp.sum(g * g, axis=0).reshape(HEADS, HID)

    @pl.when(i == 0)
    def _():
        cs_ref[...] = cs
        cq_ref[...] = cq

    @pl.when(i > 0)
    def _():
        cs_ref[...] = cs_ref[...] + cs
        cq_ref[...] = cq_ref[...] + cq


def _tc3a(S1, bias1):
    return pl.pallas_call(
        _tc3a_body,
        grid=(NB,),
        in_specs=[
            pl.BlockSpec((NQ, HEADS, BLK, PWH), lambda i: (0, 0, i, 0)),
            pl.BlockSpec((HEADS * HID,), lambda i: (0,)),
        ],
        out_specs=[
            pl.BlockSpec((BLK, HEADS * HID), lambda i: (i, 0)),
            pl.BlockSpec((HEADS, HID), lambda i: (0, 0)),
            pl.BlockSpec((HEADS, HID), lambda i: (0, 0)),
        ],
        out_shape=[
            jax.ShapeDtypeStruct((N, HEADS * HID), jnp.float32),
            jax.ShapeDtypeStruct((HEADS, HID), jnp.float32),
            jax.ShapeDtypeStruct((HEADS, HID), jnp.float32),
        ],
    )(S1, bias1)


def _tc3b_body(g_ref, cs_ref, cq_ref, gm_ref, bt_ref, w2_ref, as2_ref,
               ad2_ref, hp_ref, als_ref, ald_ref, ms_ref, md_ref):
    i = pl.program_id(0)
    mu = (cs_ref[...] / N).reshape(HEADS * HID)
    var = (cq_ref[...] / N).reshape(HEADS * HID) - mu * mu
    xn = (g_ref[...] - mu) / jnp.sqrt(var + 1e-5) * gm_ref[...] + bt_ref[...]
    h1 = jnp.where(xn > 0, xn, jnp.exp(xn) - 1.0)
    h2 = jnp.dot(h1, w2_ref[...], preferred_element_type=jnp.float32)
    for q in range(NQ):
        hp_ref[q] = _half_pad(h2[:, q * FH:(q + 1) * FH])
    als = jnp.dot(h2, as2_ref[0], preferred_element_type=jnp.float32)
    ald = jnp.dot(h2, ad2_ref[0], preferred_element_type=jnp.float32)
    als_ref[...] = als.reshape(BLK, 1)
    ald_ref[...] = ald.reshape(BLK, 1)
    vs = jnp.full((1, 128), jnp.max(als), jnp.float32)
    vd = jnp.full((1, 128), jnp.max(ald), jnp.float32)

    @pl.when(i == 0)
    def _():
        ms_ref[...] = vs
        md_ref[...] = vd

    @pl.when(i > 0)
    def _():
        ms_ref[...] = jnp.maximum(ms_ref[...], vs)
        md_ref[...] = jnp.maximum(md_ref[...], vd)


def _tc3b(G, CS, CQ, gamma1, beta1, W2, a_s2, a_d2):
    return pl.pallas_call(
        _tc3b_body,
        grid=(NB,),
        in_specs=[
            pl.BlockSpec((BLK, HEADS * HID), lambda i: (i, 0)),
            pl.BlockSpec((HEADS, HID), lambda i: (0, 0)),
            pl.BlockSpec((HEADS, HID), lambda i: (0, 0)),
            pl.BlockSpec((HEADS * HID,), lambda i: (0,)),
            pl.BlockSpec((HEADS * HID,), lambda i: (0,)),
            pl.BlockSpec((HEADS * HID, HID), lambda i: (0, 0)),
            pl.BlockSpec((1, HID), lambda i: (0, 0)),
            pl.BlockSpec((1, HID), lambda i: (0, 0)),
        ],
        out_specs=[
            pl.BlockSpec((NQ, BLK, PWH), lambda i: (0, i, 0)),
            pl.BlockSpec((BLK, 1), lambda i: (i, 0)),
            pl.BlockSpec((BLK, 1), lambda i: (i, 0)),
            pl.BlockSpec((1, 128), lambda i: (0, 0)),
            pl.BlockSpec((1, 128), lambda i: (0, 0)),
        ],
        out_shape=[
            jax.ShapeDtypeStruct((NQ, N, PWH), jnp.float32),
            jax.ShapeDtypeStruct((N, 1), jnp.float32),
            jax.ShapeDtypeStruct((N, 1), jnp.float32),
            jax.ShapeDtypeStruct((1, 128), jnp.float32),
            jax.ShapeDtypeStruct((1, 128), jnp.float32),
        ],
    )(G, CS, CQ, gamma1, beta1, W2, a_s2, a_d2)


def _tc5a_body(s_ref, b_ref, g_ref, cs_ref, cq_ref):
    i = pl.program_id(0)
    s = s_ref[...]
    tot = s[:, 0] + s[:, 1]
    num = jnp.concatenate([tot[q, :, 0:FH] for q in range(NQ)], axis=1)
    den = tot[0, :, FH:FH + 1]
    g = num / (den + 1e-16) + b_ref[...]
    g_ref[...] = g
    cs = jnp.sum(g, axis=0, keepdims=True)
    cq = jnp.sum(g * g, axis=0, keepdims=True)

    @pl.when(i == 0)
    def _():
        cs_ref[...] = cs
        cq_ref[...] = cq

    @pl.when(i > 0)
    def _():
        cs_ref[...] = cs_ref[...] + cs
        cq_ref[...] = cq_ref[...] + cq


def _tc5a(S2P, bias2):
    return pl.pallas_call(
        _tc5a_body,
        grid=(NB,),
        in_specs=[
            pl.BlockSpec((NQ, 2, BLK, PWH), lambda i: (0, 0, i, 0)),
            pl.BlockSpec((HID,), lambda i: (0,)),
        ],
        out_specs=[
            pl.BlockSpec((BLK, HID), lambda i: (i, 0)),
            pl.BlockSpec((1, HID), lambda i: (0, 0)),
            pl.BlockSpec((1, HID), lambda i: (0, 0)),
        ],
        out_shape=[
            jax.ShapeDtypeStruct((N, HID), jnp.float32),
            jax.ShapeDtypeStruct((1, HID), jnp.float32),
            jax.ShapeDtypeStruct((1, HID), jnp.float32),
        ],
    )(S2P, bias2)


def _tc5b_body(g_ref, cs_ref, cq_ref, gm_ref, bt_ref, w1_ref, b1_ref,
               w2_ref, b2_ref, o_ref):
    mu = cs_ref[0] / N
    var = cq_ref[0] / N - mu * mu
    xn = (g_ref[...] - mu) / jnp.sqrt(var + 1e-5) * gm_ref[...] + bt_ref[...]
    h = jnp.where(xn > 0, xn, jnp.exp(xn) - 1.0)
    z = jnp.maximum(
        jnp.dot(h, w1_ref[...], preferred_element_type=jnp.float32) + b1_ref[...], 0.0)
    o_ref[...] = jnp.dot(z, w2_ref[...], preferred_element_type=jnp.float32) + b2_ref[...]


def _tc5b(G2, CS2, CQ2, gamma2, beta2, cls_W1, cls_b1, w2p, b2p):
    return pl.pallas_call(
        _tc5b_body,
        grid=(NB,),
        in_specs=[
            pl.BlockSpec((BLK, HID), lambda i: (i, 0)),
            pl.BlockSpec((1, HID), lambda i: (0, 0)),
            pl.BlockSpec((1, HID), lambda i: (0, 0)),
            pl.BlockSpec((HID,), lambda i: (0,)),
            pl.BlockSpec((HID,), lambda i: (0,)),
            pl.BlockSpec((HID, 64), lambda i: (0, 0)),
            pl.BlockSpec((64,), lambda i: (0,)),
            pl.BlockSpec((64, 128), lambda i: (0, 0)),
            pl.BlockSpec((128,), lambda i: (0,)),
        ],
        out_specs=pl.BlockSpec((BLK, 128), lambda i: (i, 0)),
        out_shape=jax.ShapeDtypeStruct((N, 128), jnp.float32),
    )(G2, CS2, CQ2, gamma2, beta2, cls_W1, cls_b1, w2p, b2p)


def _prep_chunk(src_v, dstc_v, als_v, ald_v, gidx_v, exc_v, b, g, hoff, m):
    for v in range(KCH // 16):
        sl = pl.ds(v * 16, 16)
        s16 = src_v[pl.ds(g * KCH + v * 16, 16)]
        d16 = dstc_v[g, sl]
        a1 = plsc.load_gather(als_v, [s16])
        a2 = plsc.load_gather(ald_v, [d16])
        e = a1 + a2
        e = jnp.maximum(e, e * 0.2)
        exc_v[b, sl] = jnp.exp(e - m)
        gidx_v[b, sl] = s16 + hoff


def _scale_chunk(exc_v, rows_v, b):
    def vgroup(v, carry):
        ex16 = exc_v[b, pl.ds(v * 16, 16)]
        r0 = v * 16
        for rr in range(16):
            exs = ex16[rr]
            for j in range(PWH // 16):
                sl2 = pl.ds(j * 16, 16)
                rows_v[b, r0 + rr, sl2] = rows_v[b, r0 + rr, sl2] * exs
        return carry

    lax.fori_loop(0, KCH // 16, vgroup, 0)


def _edge_loop(hp, acc, src_v, dstc_v, als_v, ald_v, gidx_v, exc_v, rows_v,
               gsem, ssem, nch, hoff, m):
    def g_copy(b, g):
        return pltpu.make_async_copy(hp.at[gidx_v.at[b]], rows_v.at[b],
                                     gsem.at[b])

    def s_copy(b, g):
        return pltpu.make_async_copy(rows_v.at[b], acc.at[dstc_v.at[g]],
                                     ssem.at[b])

    _prep_chunk(src_v, dstc_v, als_v, ald_v, gidx_v, exc_v, 0, 0, hoff, m)
    g_copy(0, 0).start()

    def _one(g, b):
        nb = 1 - b

        @pl.when(g + 1 < nch)
        def _w():
            _prep_chunk(src_v, dstc_v, als_v, ald_v, gidx_v, exc_v, nb,
                        g + 1, hoff, m)
            g_copy(nb, g + 1).start()

        g_copy(b, g).wait()
        _scale_chunk(exc_v, rows_v, b)
        pltpu.sync_copy(rows_v.at[b], acc.at[dstc_v.at[g]], add=True)

    def pair_body(k, carry):
        _one(2 * k, 0)
        _one(2 * k + 1, 1)
        return carry

    lax.fori_loop(0, nch // 2, pair_body, 0)
    if nch % 2:
        _one(nch - 1, 0)


def _zero_acc(rows_v, acc, rbase):
    def zcopy(k, _):
        pltpu.sync_copy(rows_v.at[0], acc.at[pl.ds(rbase + k * KCH, KCH), :])
        return _

    lax.fori_loop(0, RZ // KCH, zcopy, 0)
    rem = RZ - (RZ // KCH) * KCH
    if rem:
        pltpu.sync_copy(rows_v.at[0, pl.ds(0, rem), :],
                        acc.at[pl.ds(rbase + (RZ // KCH) * KCH, rem), :])


def _zero_rows(rows_v):
    def zrow(r, _):
        for j in range(PWH // 16):
            rows_v[0, r, pl.ds(j * 16, 16)] = jnp.zeros((16,), jnp.float32)
        return _

    lax.fori_loop(0, KCH, zrow, 0)


def _sc_l1_body(hp, als, ald, srcr, dst3, m32, s1,
                als_v, ald_v, src_v, dstc_v, gidx_v, exc_v, rows_v, m_v,
                gsem, ssem, acc):
    c = lax.axis_index("c")
    s = lax.axis_index("s")
    rbase = s * RZ
    pltpu.sync_copy(m32, m_v)
    pltpu.sync_copy(dst3.at[s], dstc_v)
    pltpu.sync_copy(srcr.at[pl.ds(s * EPT1, EPT1)], src_v)

    for hh in range(HEADS // 2):
        hI = c * (HEADS // 2) + hh
        pltpu.sync_copy(als.at[hI], als_v)
        pltpu.sync_copy(ald.at[hI], ald_v)
        m = m_v[pl.ds(hI, 16)][0]

        for q in range(NQ):
            _zero_rows(rows_v)

            @pl.when(s < RTILES)
            def _():
                _zero_acc(rows_v, acc, rbase)

            plsc.subcore_barrier()
            hoff = (q * HEADS + hI) * NT
            _edge_loop(hp, acc, src_v, dstc_v, als_v, ald_v, gidx_v, exc_v,
                       rows_v, gsem, ssem, NCH1, hoff, m)
            plsc.subcore_barrier()

            @pl.when(s < RTILES)
            def _():
                pltpu.sync_copy(acc.at[pl.ds(rbase, RZ), :],
                                s1.at[q, hI, pl.ds(rbase, RZ), :])

            plsc.subcore_barrier()


def _sc_l1(H1flat, ALS, ALD, src, dst3, M32):
    f = pl.kernel(
        _sc_l1_body,
        out_type=jax.ShapeDtypeStruct((NQ, HEADS, NACC, PWH), jnp.float32),
        mesh=plsc.VectorSubcoreMesh(core_axis_name="c", subcore_axis_name="s"),
        compiler_params=_SC_PARAMS,
        scratch_types=[
            pltpu.VMEM((NT,), jnp.float32),
            pltpu.VMEM((NT,), jnp.float32),
            pltpu.VMEM((EPT1,), jnp.int32),
            pltpu.VMEM((NCH1, KCH), jnp.int32),
            pltpu.VMEM((2, KCH), jnp.int32),
            pltpu.VMEM((2, KCH), jnp.float32),
            pltpu.VMEM((2, KCH, PWH), jnp.float32),
            pltpu.VMEM((32,), jnp.float32),
            pltpu.SemaphoreType.DMA((2,)),
            pltpu.SemaphoreType.DMA((2,)),
            pltpu.VMEM_SHARED((NACC, PWH), jnp.float32),
        ],
    )
    return f(H1flat, ALS, ALD, src, dst3, M32)


def _sc_l2_body(hp, als, ald, srcr, dst3, m32, s2p,
                als_v, ald_v, src_v, dstc_v, gidx_v, exc_v, rows_v, m_v,
                gsem, ssem, acc):
    c = lax.axis_index("c")
    s = lax.axis_index("s")
    w = c * 16 + s
    rbase = s * RZ
    pltpu.sync_copy(m32, m_v)
    pltpu.sync_copy(dst3.at[w], dstc_v)
    pltpu.sync_copy(srcr.at[pl.ds(w * EPT2, EPT2)], src_v)
    pltpu.sync_copy(als, als_v)
    pltpu.sync_copy(ald, ald_v)
    m = m_v[pl.ds(0, 16)][0]

    for q in range(NQ):
        _zero_rows(rows_v)

        @pl.when(s < RTILES)
        def _():
            _zero_acc(rows_v, acc, rbase)

        plsc.subcore_barrier()
        hoff = q * N
        _edge_loop(hp, acc, src_v, dstc_v, als_v, ald_v, gidx_v, exc_v,
                   rows_v, gsem, ssem, NCH2, hoff, m)
        plsc.subcore_barrier()

        @pl.when(s < RTILES)
        def _():
            pltpu.sync_copy(acc.at[pl.ds(rbase, RZ), :],
                            s2p.at[q, c, pl.ds(rbase, RZ), :])

        plsc.subcore_barrier()


def _sc_l2(H2flat, ALS2, ALD2, src, dst3, M32):
    f = pl.kernel(
        _sc_l2_body,
        out_type=jax.ShapeDtypeStruct((NQ, 2, NACC, PWH), jnp.float32),
        mesh=plsc.VectorSubcoreMesh(core_axis_name="c", subcore_axis_name="s"),
        compiler_params=_SC_PARAMS,
        scratch_types=[
            pltpu.VMEM((N,), jnp.float32),
            pltpu.VMEM((N,), jnp.float32),
            pltpu.VMEM((EPT2,), jnp.int32),
            pltpu.VMEM((NCH2, KCH), jnp.int32),
            pltpu.VMEM((2, KCH), jnp.int32),
            pltpu.VMEM((2, KCH), jnp.float32),
            pltpu.VMEM((2, KCH, PWH), jnp.float32),
            pltpu.VMEM((32,), jnp.float32),
            pltpu.SemaphoreType.DMA((2,)),
            pltpu.SemaphoreType.DMA((2,)),
            pltpu.VMEM_SHARED((NACC, PWH), jnp.float32),
        ],
    )
    return f(H2flat, ALS2, ALD2, src, dst3, M32)


def _leaky(x):
    return jnp.where(x > 0, x, 0.2 * x)


def kernel(x, edge_index, W1, att_src1, att_dst1, bias1, gamma1, beta1,
           W2, att_src2, att_dst2, bias2, gamma2, beta2,
           cls_W1, cls_b1, cls_W2, cls_b2):
    src = edge_index[0]
    dst = edge_index[1]
    dst31 = dst.reshape(16, NCH1, KCH)
    dst32 = dst.reshape(32, NCH2, KCH)

    xp = jnp.pad(x, ((0, NT - N), (0, 0)))
    H1p, ALS, ALD, MS, MD = _tc1(xp, W1, att_src1, att_dst1)
    M1 = _leaky(MS[0] + MD[0])
    M32 = jnp.concatenate([M1, jnp.zeros((24,), jnp.float32)])

    S1 = _sc_l1(H1p.reshape(NQ * HEADS * NT, PWH), ALS, ALD,
                src, dst31, M32)

    G, CS, CQ = _tc3a(S1, bias1)
    H2p, ALS2, ALD2, MS2, MD2 = _tc3b(G, CS, CQ, gamma1, beta1, W2,
                                      att_src2, att_dst2)
    M2 = _leaky(MS2[0, 0] + MD2[0, 0])
    M232 = jnp.full((32,), M2, jnp.float32)

    S2P = _sc_l2(H2p.reshape(NQ * N, PWH), ALS2.reshape(N), ALD2.reshape(N),
                 src, dst32, M232)

    G2, CS2, CQ2 = _tc5a(S2P, bias2)
    w2p = jnp.pad(cls_W2, ((0, 0), (0, 126)))
    b2p = jnp.pad(cls_b2, (0, 126))
    out = _tc5b(G2, CS2, CQ2, gamma2, beta2, cls_W1, cls_b1, w2p, b2p)
    return out[:, :2]


# speedup vs baseline: 25.4581x; 1.1447x over previous
import jax
import jax.numpy as jnp
from jax import lax
from jax.experimental import pallas as pl
from jax.experimental.pallas import tpu as pltpu
from jax.experimental.pallas import tpu_sc as plsc

N = 10000
E = 320000
HEADS = 8
HID = 128
FH = 64
NQ = 2
PWH = 80
BLK = 400
NB = N // BLK
KCH = 80
EPT1 = E // 16
NCH1 = EPT1 // KCH
EPT2 = E // 32
NCH2 = EPT2 // KCH
NACC = 10000
RTILES = 10
RZ = NACC // RTILES
NT = 10240
BLKP = 512
NBP = NT // BLKP

_SC_PARAMS = pltpu.CompilerParams(needs_layout_passes=False,
                                  use_tc_tiling_on_sc=False)


def _half_pad(mat):
    rows = mat.shape[0]
    return jnp.concatenate(
        [mat, jnp.ones((rows, 1), jnp.float32),
         jnp.zeros((rows, PWH - FH - 1), jnp.float32)],
        axis=1)


def _tc1_body(x_ref, w_ref, as_ref, ad_ref, hp_ref, als_ref, ald_ref,
              ms_ref, md_ref):
    i = pl.program_id(0)
    hb = jnp.dot(x_ref[...], w_ref[...], preferred_element_type=jnp.float32)
    for q in range(NQ):
        for j in range(HEADS):
            base = j * HID + q * FH
            hp_ref[q, j] = _half_pad(hb[:, base:base + FH])
    hbr = hb.reshape(BLKP, HEADS, HID)
    als = jnp.sum(hbr * as_ref[...][None], axis=2)
    ald = jnp.sum(hbr * ad_ref[...][None], axis=2)
    als_ref[...] = als.T
    ald_ref[...] = ald.T
    vs = jnp.max(als, axis=0, keepdims=True)
    vd = jnp.max(ald, axis=0, keepdims=True)

    @pl.when(i == 0)
    def _():
        ms_ref[...] = vs
        md_ref[...] = vd

    @pl.when(i > 0)
    def _():
        ms_ref[...] = jnp.maximum(ms_ref[...], vs)
        md_ref[...] = jnp.maximum(md_ref[...], vd)


def _tc1(xp, W1, a_s, a_d):
    return pl.pallas_call(
        _tc1_body,
        grid=(NBP,),
        in_specs=[
            pl.BlockSpec((BLKP, HID), lambda i: (i, 0)),
            pl.BlockSpec((HID, HEADS * HID), lambda i: (0, 0)),
            pl.BlockSpec((HEADS, HID), lambda i: (0, 0)),
            pl.BlockSpec((HEADS, HID), lambda i: (0, 0)),
        ],
        out_specs=[
            pl.BlockSpec((NQ, HEADS, BLKP, PWH), lambda i: (0, 0, i, 0)),
            pl.BlockSpec((HEADS, BLKP), lambda i: (0, i)),
            pl.BlockSpec((HEADS, BLKP), lambda i: (0, i)),
            pl.BlockSpec((1, HEADS), lambda i: (0, 0)),
            pl.BlockSpec((1, HEADS), lambda i: (0, 0)),
        ],
        out_shape=[
            jax.ShapeDtypeStruct((NQ, HEADS, NT, PWH), jnp.float32),
            jax.ShapeDtypeStruct((HEADS, NT), jnp.float32),
            jax.ShapeDtypeStruct((HEADS, NT), jnp.float32),
            jax.ShapeDtypeStruct((1, HEADS), jnp.float32),
            jax.ShapeDtypeStruct((1, HEADS), jnp.float32),
        ],
    )(xp, W1, a_s, a_d)


def _tc3a_body(s_ref, b_ref, g_ref, cs_ref, cq_ref):
    i = pl.program_id(0)
    s = s_ref[...]
    num = jnp.concatenate([s[q, :, :, 0:FH] for q in range(NQ)], axis=2)
    den = s[0, :, :, FH:FH + 1]
    g = num / (den + 1e-16)
    g = jnp.transpose(g, (1, 0, 2)).reshape(BLK, HEADS * HID) + b_ref[...]
    g_ref[...] = g
    cs = jnp.sum(g, axis=0).reshape(HEADS, HID)
    cq = jnp.sum(g * g, axis=0).reshape(HEADS, HID)

    @pl.when(i == 0)
    def _():
        cs_ref[...] = cs
        cq_ref[...] = cq

    @pl.when(i > 0)
    def _():
        cs_ref[...] = cs_ref[...] + cs
        cq_ref[...] = cq_ref[...] + cq


def _tc3a(S1, bias1):
    return pl.pallas_call(
        _tc3a_body,
        grid=(NB,),
        in_specs=[
            pl.BlockSpec((NQ, HEADS, BLK, PWH), lambda i: (0, 0, i, 0)),
            pl.BlockSpec((HEADS * HID,), lambda i: (0,)),
        ],
        out_specs=[
            pl.BlockSpec((BLK, HEADS * HID), lambda i: (i, 0)),
            pl.BlockSpec((HEADS, HID), lambda i: (0, 0)),
            pl.BlockSpec((HEADS, HID), lambda i: (0, 0)),
        ],
        out_shape=[
            jax.ShapeDtypeStruct((N, HEADS * HID), jnp.float32),
            jax.ShapeDtypeStruct((HEADS, HID), jnp.float32),
            jax.ShapeDtypeStruct((HEADS, HID), jnp.float32),
        ],
    )(S1, bias1)


def _tc3b_body(g_ref, cs_ref, cq_ref, gm_ref, bt_ref, w2_ref, as2_ref,
               ad2_ref, hp_ref, als_ref, ald_ref, ms_ref, md_ref):
    i = pl.program_id(0)
    mu = (cs_ref[...] / N).reshape(HEADS * HID)
    var = (cq_ref[...] / N).reshape(HEADS * HID) - mu * mu
    xn = (g_ref[...] - mu) / jnp.sqrt(var + 1e-5) * gm_ref[...] + bt_ref[...]
    h1 = jnp.where(xn > 0, xn, jnp.exp(xn) - 1.0)
    h2 = jnp.dot(h1, w2_ref[...], preferred_element_type=jnp.float32)
    for q in range(NQ):
        hp_ref[q] = _half_pad(h2[:, q * FH:(q + 1) * FH])
    als = jnp.dot(h2, as2_ref[0], preferred_element_type=jnp.float32)
    ald = jnp.dot(h2, ad2_ref[0], preferred_element_type=jnp.float32)
    als_ref[...] = als.reshape(BLK, 1)
    ald_ref[...] = ald.reshape(BLK, 1)
    vs = jnp.full((1, 128), jnp.max(als), jnp.float32)
    vd = jnp.full((1, 128), jnp.max(ald), jnp.float32)

    @pl.when(i == 0)
    def _():
        ms_ref[...] = vs
        md_ref[...] = vd

    @pl.when(i > 0)
    def _():
        ms_ref[...] = jnp.maximum(ms_ref[...], vs)
        md_ref[...] = jnp.maximum(md_ref[...], vd)


def _tc3b(G, CS, CQ, gamma1, beta1, W2, a_s2, a_d2):
    return pl.pallas_call(
        _tc3b_body,
        grid=(NB,),
        in_specs=[
            pl.BlockSpec((BLK, HEADS * HID), lambda i: (i, 0)),
            pl.BlockSpec((HEADS, HID), lambda i: (0, 0)),
            pl.BlockSpec((HEADS, HID), lambda i: (0, 0)),
            pl.BlockSpec((HEADS * HID,), lambda i: (0,)),
            pl.BlockSpec((HEADS * HID,), lambda i: (0,)),
            pl.BlockSpec((HEADS * HID, HID), lambda i: (0, 0)),
            pl.BlockSpec((1, HID), lambda i: (0, 0)),
            pl.BlockSpec((1, HID), lambda i: (0, 0)),
        ],
        out_specs=[
            pl.BlockSpec((NQ, BLK, PWH), lambda i: (0, i, 0)),
            pl.BlockSpec((BLK, 1), lambda i: (i, 0)),
            pl.BlockSpec((BLK, 1), lambda i: (i, 0)),
            pl.BlockSpec((1, 128), lambda i: (0, 0)),
            pl.BlockSpec((1, 128), lambda i: (0, 0)),
        ],
        out_shape=[
            jax.ShapeDtypeStruct((NQ, N, PWH), jnp.float32),
            jax.ShapeDtypeStruct((N, 1), jnp.float32),
            jax.ShapeDtypeStruct((N, 1), jnp.float32),
            jax.ShapeDtypeStruct((1, 128), jnp.float32),
            jax.ShapeDtypeStruct((1, 128), jnp.float32),
        ],
    )(G, CS, CQ, gamma1, beta1, W2, a_s2, a_d2)


def _tc5a_body(s_ref, b_ref, g_ref, cs_ref, cq_ref):
    i = pl.program_id(0)
    s = s_ref[...]
    tot = s[:, 0] + s[:, 1]
    num = jnp.concatenate([tot[q, :, 0:FH] for q in range(NQ)], axis=1)
    den = tot[0, :, FH:FH + 1]
    g = num / (den + 1e-16) + b_ref[...]
    g_ref[...] = g
    cs = jnp.sum(g, axis=0, keepdims=True)
    cq = jnp.sum(g * g, axis=0, keepdims=True)

    @pl.when(i == 0)
    def _():
        cs_ref[...] = cs
        cq_ref[...] = cq

    @pl.when(i > 0)
    def _():
        cs_ref[...] = cs_ref[...] + cs
        cq_ref[...] = cq_ref[...] + cq


def _tc5a(S2P, bias2):
    return pl.pallas_call(
        _tc5a_body,
        grid=(NB,),
        in_specs=[
            pl.BlockSpec((NQ, 2, BLK, PWH), lambda i: (0, 0, i, 0)),
            pl.BlockSpec((HID,), lambda i: (0,)),
        ],
        out_specs=[
            pl.BlockSpec((BLK, HID), lambda i: (i, 0)),
            pl.BlockSpec((1, HID), lambda i: (0, 0)),
            pl.BlockSpec((1, HID), lambda i: (0, 0)),
        ],
        out_shape=[
            jax.ShapeDtypeStruct((N, HID), jnp.float32),
            jax.ShapeDtypeStruct((1, HID), jnp.float32),
            jax.ShapeDtypeStruct((1, HID), jnp.float32),
        ],
    )(S2P, bias2)


def _tc5b_body(g_ref, cs_ref, cq_ref, gm_ref, bt_ref, w1_ref, b1_ref,
               w2_ref, b2_ref, o_ref):
    mu = cs_ref[0] / N
    var = cq_ref[0] / N - mu * mu
    xn = (g_ref[...] - mu) / jnp.sqrt(var + 1e-5) * gm_ref[...] + bt_ref[...]
    h = jnp.where(xn > 0, xn, jnp.exp(xn) - 1.0)
    z = jnp.maximum(
        jnp.dot(h, w1_ref[...], preferred_element_type=jnp.float32) + b1_ref[...], 0.0)
    o_ref[...] = jnp.dot(z, w2_ref[...], preferred_element_type=jnp.float32) + b2_ref[...]


def _tc5b(G2, CS2, CQ2, gamma2, beta2, cls_W1, cls_b1, w2p, b2p):
    return pl.pallas_call(
        _tc5b_body,
        grid=(NB,),
        in_specs=[
            pl.BlockSpec((BLK, HID), lambda i: (i, 0)),
            pl.BlockSpec((1, HID), lambda i: (0, 0)),
            pl.BlockSpec((1, HID), lambda i: (0, 0)),
            pl.BlockSpec((HID,), lambda i: (0,)),
            pl.BlockSpec((HID,), lambda i: (0,)),
            pl.BlockSpec((HID, 64), lambda i: (0, 0)),
            pl.BlockSpec((64,), lambda i: (0,)),
            pl.BlockSpec((64, 128), lambda i: (0, 0)),
            pl.BlockSpec((128,), lambda i: (0,)),
        ],
        out_specs=pl.BlockSpec((BLK, 128), lambda i: (i, 0)),
        out_shape=jax.ShapeDtypeStruct((N, 128), jnp.float32),
    )(G2, CS2, CQ2, gamma2, beta2, cls_W1, cls_b1, w2p, b2p)


def _prep_chunk(src_v, dstc_v, als_v, ald_v, gidx_v, exc_v, b, g, hoff, m):
    def vstep(v, carry):
        sl = pl.ds(v * 16, 16)
        s16 = src_v[pl.ds(g * KCH + v * 16, 16)]
        d16 = dstc_v[g, sl]
        a1 = plsc.load_gather(als_v, [s16])
        a2 = plsc.load_gather(ald_v, [d16])
        e = a1 + a2
        e = jnp.maximum(e, e * 0.2)
        exc_v[b, sl] = jnp.exp(e - m)
        gidx_v[b, sl] = s16 + hoff
        return carry

    lax.fori_loop(0, KCH // 16, vstep, 0)


def _scale_chunk(exc_v, rows_v, b):
    def vgroup(v, carry):
        ex16 = exc_v[b, pl.ds(v * 16, 16)]
        r0 = v * 16
        for rr in range(16):
            exs = ex16[rr]
            for j in range(PWH // 16):
                sl2 = pl.ds(j * 16, 16)
                rows_v[b, r0 + rr, sl2] = rows_v[b, r0 + rr, sl2] * exs
        return carry

    lax.fori_loop(0, KCH // 16, vgroup, 0)


def _edge_loop(hp, acc, src_v, dstc_v, als_v, ald_v, gidx_v, exc_v, rows_v,
               gsem, ssem, nch, hoff, m):
    def g_copy(b, g):
        return pltpu.make_async_copy(hp.at[gidx_v.at[b]], rows_v.at[b],
                                     gsem.at[b])

    def s_copy(b, g):
        return pltpu.make_async_copy(rows_v.at[b], acc.at[dstc_v.at[g]],
                                     ssem.at[b])

    _prep_chunk(src_v, dstc_v, als_v, ald_v, gidx_v, exc_v, 0, 0, hoff, m)
    g_copy(0, 0).start()

    def _one(g, b):
        nb = (b + 1) % 3

        @pl.when(g >= 2)
        def _wd():
            s_copy(nb, g - 2).wait()

        @pl.when(g + 1 < nch)
        def _wg():
            _prep_chunk(src_v, dstc_v, als_v, ald_v, gidx_v, exc_v, nb,
                        g + 1, hoff, m)
            g_copy(nb, g + 1).start()

        g_copy(b, g).wait()
        _scale_chunk(exc_v, rows_v, b)
        s_copy(b, g).start(add=True)

    def trip_body(k, carry):
        _one(3 * k, 0)
        _one(3 * k + 1, 1)
        _one(3 * k + 2, 2)
        return carry

    lax.fori_loop(0, nch // 3, trip_body, 0)
    for i in range(nch % 3):
        _one((nch // 3) * 3 + i, i)
    s_copy((nch - 2) % 3, nch - 2).wait()
    s_copy((nch - 1) % 3, nch - 1).wait()


def _zero_acc(rows_v, acc, rbase):
    def zcopy(k, _):
        pltpu.sync_copy(rows_v.at[0], acc.at[pl.ds(rbase + k * KCH, KCH), :])
        return _

    lax.fori_loop(0, RZ // KCH, zcopy, 0)
    rem = RZ - (RZ // KCH) * KCH
    if rem:
        pltpu.sync_copy(rows_v.at[0, pl.ds(0, rem), :],
                        acc.at[pl.ds(rbase + (RZ // KCH) * KCH, rem), :])


def _zero_rows(rows_v):
    def zrow(r, _):
        for j in range(PWH // 16):
            rows_v[0, r, pl.ds(j * 16, 16)] = jnp.zeros((16,), jnp.float32)
        return _

    lax.fori_loop(0, KCH, zrow, 0)


def _sc_l1_body(hp, als, ald, srcr, dst3, m32, s1,
                als_v, ald_v, src_v, dstc_v, gidx_v, exc_v, rows_v, m_v,
                gsem, ssem, acc):
    c = lax.axis_index("c")
    s = lax.axis_index("s")
    rbase = s * RZ
    pltpu.sync_copy(m32, m_v)
    pltpu.sync_copy(dst3.at[s], dstc_v)
    pltpu.sync_copy(srcr.at[pl.ds(s * EPT1, EPT1)], src_v)

    def pass_body(p, carry):
        hh = p // NQ
        q = p - hh * NQ
        hI = c * (HEADS // 2) + hh
        pltpu.sync_copy(als.at[hI], als_v)
        pltpu.sync_copy(ald.at[hI], ald_v)
        m = m_v[pl.ds(hI, 16)][0]
        _zero_rows(rows_v)

        @pl.when(s < RTILES)
        def _z():
            _zero_acc(rows_v, acc, rbase)

        plsc.subcore_barrier()
        hoff = (q * HEADS + hI) * NT
        _edge_loop(hp, acc, src_v, dstc_v, als_v, ald_v, gidx_v, exc_v,
                   rows_v, gsem, ssem, NCH1, hoff, m)
        plsc.subcore_barrier()

        @pl.when(s < RTILES)
        def _r():
            pltpu.sync_copy(acc.at[pl.ds(rbase, RZ), :],
                            s1.at[q, hI, pl.ds(rbase, RZ), :])

        plsc.subcore_barrier()
        return carry

    lax.fori_loop(0, (HEADS // 2) * NQ, pass_body, 0)


def _sc_l1(H1flat, ALS, ALD, src, dst3, M32):
    f = pl.kernel(
        _sc_l1_body,
        out_type=jax.ShapeDtypeStruct((NQ, HEADS, NACC, PWH), jnp.float32),
        mesh=plsc.VectorSubcoreMesh(core_axis_name="c", subcore_axis_name="s"),
        compiler_params=_SC_PARAMS,
        scratch_types=[
            pltpu.VMEM((NT,), jnp.float32),
            pltpu.VMEM((NT,), jnp.float32),
            pltpu.VMEM((EPT1,), jnp.int32),
            pltpu.VMEM((NCH1, KCH), jnp.int32),
            pltpu.VMEM((3, KCH), jnp.int32),
            pltpu.VMEM((3, KCH), jnp.float32),
            pltpu.VMEM((3, KCH, PWH), jnp.float32),
            pltpu.VMEM((32,), jnp.float32),
            pltpu.SemaphoreType.DMA((3,)),
            pltpu.SemaphoreType.DMA((3,)),
            pltpu.VMEM_SHARED((NACC, PWH), jnp.float32),
        ],
    )
    return f(H1flat, ALS, ALD, src, dst3, M32)


def _sc_l2_body(hp, als, ald, srcr, dst3, m32, s2p,
                als_v, ald_v, src_v, dstc_v, gidx_v, exc_v, rows_v, m_v,
                gsem, ssem, acc):
    c = lax.axis_index("c")
    s = lax.axis_index("s")
    w = c * 16 + s
    rbase = s * RZ
    pltpu.sync_copy(m32, m_v)
    pltpu.sync_copy(dst3.at[w], dstc_v)
    pltpu.sync_copy(srcr.at[pl.ds(w * EPT2, EPT2)], src_v)
    pltpu.sync_copy(als, als_v)
    pltpu.sync_copy(ald, ald_v)
    m = m_v[pl.ds(0, 16)][0]

    for q in range(NQ):
        _zero_rows(rows_v)

        @pl.when(s < RTILES)
        def _():
            _zero_acc(rows_v, acc, rbase)

        plsc.subcore_barrier()
        hoff = q * N
        _edge_loop(hp, acc, src_v, dstc_v, als_v, ald_v, gidx_v, exc_v,
                   rows_v, gsem, ssem, NCH2, hoff, m)
        plsc.subcore_barrier()

        @pl.when(s < RTILES)
        def _():
            pltpu.sync_copy(acc.at[pl.ds(rbase, RZ), :],
                            s2p.at[q, c, pl.ds(rbase, RZ), :])

        plsc.subcore_barrier()


def _sc_l2(H2flat, ALS2, ALD2, src, dst3, M32):
    f = pl.kernel(
        _sc_l2_body,
        out_type=jax.ShapeDtypeStruct((NQ, 2, NACC, PWH), jnp.float32),
        mesh=plsc.VectorSubcoreMesh(core_axis_name="c", subcore_axis_name="s"),
        compiler_params=_SC_PARAMS,
        scratch_types=[
            pltpu.VMEM((N,), jnp.float32),
            pltpu.VMEM((N,), jnp.float32),
            pltpu.VMEM((EPT2,), jnp.int32),
            pltpu.VMEM((NCH2, KCH), jnp.int32),
            pltpu.VMEM((3, KCH), jnp.int32),
            pltpu.VMEM((3, KCH), jnp.float32),
            pltpu.VMEM((3, KCH, PWH), jnp.float32),
            pltpu.VMEM((32,), jnp.float32),
            pltpu.SemaphoreType.DMA((3,)),
            pltpu.SemaphoreType.DMA((3,)),
            pltpu.VMEM_SHARED((NACC, PWH), jnp.float32),
        ],
    )
    return f(H2flat, ALS2, ALD2, src, dst3, M32)


def _leaky(x):
    return jnp.where(x > 0, x, 0.2 * x)


def kernel(x, edge_index, W1, att_src1, att_dst1, bias1, gamma1, beta1,
           W2, att_src2, att_dst2, bias2, gamma2, beta2,
           cls_W1, cls_b1, cls_W2, cls_b2):
    src = edge_index[0]
    dst = edge_index[1]
    dst31 = dst.reshape(16, NCH1, KCH)
    dst32 = dst.reshape(32, NCH2, KCH)

    xp = jnp.pad(x, ((0, NT - N), (0, 0)))
    H1p, ALS, ALD, MS, MD = _tc1(xp, W1, att_src1, att_dst1)
    M1 = _leaky(MS[0] + MD[0])
    M32 = jnp.concatenate([M1, jnp.zeros((24,), jnp.float32)])

    S1 = _sc_l1(H1p.reshape(NQ * HEADS * NT, PWH), ALS, ALD,
                src, dst31, M32)

    G, CS, CQ = _tc3a(S1, bias1)
    H2p, ALS2, ALD2, MS2, MD2 = _tc3b(G, CS, CQ, gamma1, beta1, W2,
                                      att_src2, att_dst2)
    M2 = _leaky(MS2[0, 0] + MD2[0, 0])
    M232 = jnp.full((32,), M2, jnp.float32)

    S2P = _sc_l2(H2p.reshape(NQ * N, PWH), ALS2.reshape(N), ALD2.reshape(N),
                 src, dst32, M232)

    G2, CS2, CQ2 = _tc5a(S2P, bias2)
    w2p = jnp.pad(cls_W2, ((0, 0), (0, 126)))
    b2p = jnp.pad(cls_b2, (0, 126))
    out = _tc5b(G2, CS2, CQ2, gamma2, beta2, cls_W1, cls_b1, w2p, b2p)
    return out[:, :2]
